# baseline (speedup 1.0000x reference)
import functools

import jax
import jax.numpy as jnp
from jax import lax
from jax.experimental import pallas as pl
from jax.experimental.pallas import tpu as pltpu
from jax.experimental.pallas import tpu_sc as plsc

N_NODES = 10000
N_REL = 4
D_IN = 256
D_OUT = 256
DW = 272
E = 40000
LANES = 16
NC = 2
NS = 16
NW = NC * NS
E_WORK = 1280
E_PAD = NW * E_WORK
LCAP = E_WORK + 128
SEG = E_WORK
CHUNK = 128
KMAX = SEG // CHUNK
HALF = N_NODES // NC
ACC_ROWS = 5024
OUT_UNIT = 40
N_UNITS = HALF // OUT_UNIT


def _scalar(v16):
    return lax.reduce_max(v16, axes=(0,))


def _router_body(src_hbm, dst_hbm, rsrc_hbm, rldst_hbm, rcnt_hbm,
                 sv_all, dv_all, l0s, l0d, l1s, l1d, cbuf):
    i32 = jnp.int32
    cid = lax.axis_index("c").astype(i32)
    sid = lax.axis_index("s").astype(i32)
    w = cid * i32(NS) + sid

    for r in range(N_REL):
        r = i32(r)
        ebase = w * i32(E_WORK)
        pltpu.sync_copy(src_hbm.at[r, pl.ds(ebase, E_WORK)], sv_all)
        pltpu.sync_copy(dst_hbm.at[r, pl.ds(ebase, E_WORK)], dv_all)

        def step(i, carry):
            c0, c1 = carry
            i = i.astype(jnp.int32)
            sv = sv_all[pl.ds(i * LANES, LANES)]
            dv = dv_all[pl.ds(i * LANES, LANES)]
            m0 = dv < i32(HALF)
            m1 = (dv >= i32(HALF)) & (dv < i32(N_NODES))
            inc0 = plsc.cumsum(jnp.where(m0, i32(1), i32(0)))
            inc1 = plsc.cumsum(jnp.where(m1, i32(1), i32(0)))
            pos0 = jnp.maximum(c0 + inc0 - i32(1), i32(0))
            pos1 = jnp.maximum(c1 + inc1 - i32(1), i32(0))
            plsc.store_scatter(l0s, [pos0], sv, mask=m0)
            plsc.store_scatter(l0d, [pos0], dv, mask=m0)
            plsc.store_scatter(l1s, [pos1], sv, mask=m1)
            plsc.store_scatter(l1d, [pos1], dv - i32(HALF), mask=m1)
            n0 = _scalar(inc0)
            n1 = _scalar(inc1)
            return c0 + n0, c1 + n1

        c0, c1 = pl.loop(i32(0), i32(E_WORK // LANES),
                         init_carry=(i32(0), i32(0)))(step)

        trash = i32(HALF) + (w % i32(NS))
        for t in range(8):
            off = i32(t * LANES)
            zs = jnp.zeros((LANES,), jnp.int32)
            ts = jnp.full((LANES,), 0, jnp.int32) + trash
            l0s[pl.ds(c0 + off, LANES)] = zs
            l0d[pl.ds(c0 + off, LANES)] = ts
            l1s[pl.ds(c1 + off, LANES)] = zs
            l1d[pl.ds(c1 + off, LANES)] = ts
        c0p = ((c0 + i32(127)) // i32(128)) * i32(128)
        c1p = ((c1 + i32(127)) // i32(128)) * i32(128)

        @pl.loop(i32(0), i32(KMAX))
        def _f(k):
            k = k.astype(jnp.int32)

            @pl.when(k * i32(CHUNK) < c0p)
            def _():
                pltpu.sync_copy(l0s.at[pl.ds(k * CHUNK, CHUNK)],
                                rsrc_hbm.at[r, i32(0), w, k])
                pltpu.sync_copy(l0d.at[pl.ds(k * CHUNK, CHUNK)],
                                rldst_hbm.at[r, i32(0), w, k])

            @pl.when(k * i32(CHUNK) < c1p)
            def _():
                pltpu.sync_copy(l1s.at[pl.ds(k * CHUNK, CHUNK)],
                                rsrc_hbm.at[r, i32(1), w, k])
                pltpu.sync_copy(l1d.at[pl.ds(k * CHUNK, CHUNK)],
                                rldst_hbm.at[r, i32(1), w, k])

        cbuf[pl.ds(i32(0), LANES)] = jnp.zeros((LANES,), jnp.int32) + c0p
        pltpu.sync_copy(cbuf, rcnt_hbm.at[r, i32(0), w])
        cbuf[pl.ds(i32(0), LANES)] = jnp.zeros((LANES,), jnp.int32) + c1p
        pltpu.sync_copy(cbuf, rcnt_hbm.at[r, i32(1), w])


@jax.jit
def _route(src, dst):
    mesh = plsc.VectorSubcoreMesh(core_axis_name="c", subcore_axis_name="s")
    f = pl.kernel(
        _router_body,
        out_type=(
            jax.ShapeDtypeStruct((N_REL, NC, NW, KMAX, CHUNK), jnp.int32),
            jax.ShapeDtypeStruct((N_REL, NC, NW, KMAX, CHUNK), jnp.int32),
            jax.ShapeDtypeStruct((N_REL, NC, NW, LANES), jnp.int32),
        ),
        mesh=mesh,
        scratch_types=[
            pltpu.VMEM((E_WORK,), jnp.int32),
            pltpu.VMEM((E_WORK,), jnp.int32),
            pltpu.VMEM((LCAP,), jnp.int32),
            pltpu.VMEM((LCAP,), jnp.int32),
            pltpu.VMEM((LCAP,), jnp.int32),
            pltpu.VMEM((LCAP,), jnp.int32),
            pltpu.VMEM((LANES,), jnp.int32),
        ],
        compiler_params=pltpu.CompilerParams(use_tc_tiling_on_sc=False,
                                             needs_layout_passes=False),
    )
    return f(src, dst)


def _agg_body(xa_hbm, rsrc_hbm, rldst_hbm, rcnt_hbm, out_hbm,
              acc_sh, sidx, lidx, rows_v, zrow_v, cbuf, sem):
    i32 = jnp.int32
    cid = lax.axis_index("c").astype(i32)
    sid = lax.axis_index("s").astype(i32)
    base_dst = cid * i32(HALF)

    @pl.loop(i32(0), i32(DW // LANES))
    def _zb(j):
        j = j.astype(jnp.int32)
        for i in range(8):
            zrow_v[i, pl.ds(j * LANES, LANES)] = jnp.zeros((LANES,), jnp.float32)

    for r in range(N_REL):
        r = i32(r)

        @pl.loop(i32(0), i32((ACC_ROWS // 8 + NS - 1) // NS))
        def _z(k):
            zu = sid + k.astype(jnp.int32) * i32(NS)

            @pl.when(zu < i32(ACC_ROWS // 8))
            def _():
                pltpu.sync_copy(zrow_v, acc_sh.at[pl.ds(zu * 8, 8)])

        plsc.subcore_barrier()

        for seg in range(2):
            w = sid + i32(seg * NS)
            pltpu.sync_copy(rcnt_hbm.at[r, cid, w], cbuf)
            n = _scalar(cbuf[pl.ds(i32(0), LANES)])
            pltpu.sync_copy(rsrc_hbm.at[r, cid, w], sidx)
            pltpu.sync_copy(rldst_hbm.at[r, cid, w], lidx)

            @pl.loop(i32(0), i32(KMAX))
            def _e(k):
                k = k.astype(jnp.int32)

                @pl.when(k * i32(CHUNK) < n)
                def _():
                    pltpu.async_copy(xa_hbm.at[sidx.at[k]], rows_v,
                                     sem).wait()
                    pltpu.sync_copy(rows_v, acc_sh.at[lidx.at[k]], add=True)

        plsc.subcore_barrier()

        @pl.loop(i32(0), i32((N_UNITS + NS - 1) // NS))
        def _w(k):
            u = sid + k.astype(jnp.int32) * i32(NS)

            @pl.when(u < i32(N_UNITS))
            def _():
                pltpu.sync_copy(
                    acc_sh.at[pl.ds(u * OUT_UNIT, OUT_UNIT)],
                    out_hbm.at[r, pl.ds(base_dst + u * OUT_UNIT, OUT_UNIT)])

        plsc.subcore_barrier()


@jax.jit
def _sc_aggregate(xa, rsrc, rldst, rcnt):
    mesh = plsc.VectorSubcoreMesh(core_axis_name="c", subcore_axis_name="s")
    f = pl.kernel(
        _agg_body,
        out_type=jax.ShapeDtypeStruct((N_REL, N_NODES, DW), jnp.float32),
        mesh=mesh,
        scratch_types=[
            pltpu.VMEM_SHARED((ACC_ROWS, DW), jnp.float32),
            pltpu.VMEM((KMAX, CHUNK), jnp.int32),
            pltpu.VMEM((KMAX, CHUNK), jnp.int32),
            pltpu.VMEM((CHUNK, DW), jnp.float32),
            pltpu.VMEM((8, DW), jnp.float32),
            pltpu.VMEM((LANES,), jnp.int32),
            pltpu.SemaphoreType.DMA,
        ],
        compiler_params=pltpu.CompilerParams(use_tc_tiling_on_sc=False,
                                             needs_layout_passes=False),
    )
    return f(xa, rsrc, rldst, rcnt)


def _wcat_body(coeff_ref, w2d_ref, wself_ref, o_ref):
    wall = jnp.dot(coeff_ref[...], w2d_ref[...],
                   preferred_element_type=jnp.float32,
                   precision=jax.lax.Precision.HIGHEST)
    o_ref[: N_REL * D_IN, :] = wall.reshape(N_REL * D_IN, D_OUT)
    o_ref[N_REL * D_IN:, :] = wself_ref[...]


def _combine_wcat(coeff_mat, w2d, W_self):
    return pl.pallas_call(
        _wcat_body,
        out_shape=jax.ShapeDtypeStruct(((N_REL + 1) * D_IN, D_OUT), jnp.float32),
    )(coeff_mat, w2d, W_self)


def _z0(i):
    return i * 0


BM = 400


def _tc_body(s_ref, x_ref, wcat_ref, bias_ref, o_ref):
    parts = []
    for r in range(N_REL):
        cnt = s_ref[r, :, D_IN:D_IN + 1]
        sm = s_ref[r, :, :D_IN]
        parts.append(jnp.where(cnt > 0, sm / jnp.maximum(cnt, 1.0), 0.0))
    parts.append(x_ref[...])
    xin = jnp.concatenate(parts, axis=1)
    acc = jnp.dot(xin, wcat_ref[...], preferred_element_type=jnp.float32)
    o_ref[...] = jnp.maximum(acc + bias_ref[...], 0.0)


def _tc_combine(s, x, wcat, bias2d):
    grid = (N_NODES // BM,)
    return pl.pallas_call(
        _tc_body,
        grid=grid,
        in_specs=[
            pl.BlockSpec((N_REL, BM, DW), lambda i: (_z0(i), i, _z0(i))),
            pl.BlockSpec((BM, D_IN), lambda i: (i, _z0(i))),
            pl.BlockSpec(((N_REL + 1) * D_IN, D_OUT),
                         lambda i: (_z0(i), _z0(i))),
            pl.BlockSpec((1, D_OUT), lambda i: (_z0(i), _z0(i))),
        ],
        out_specs=pl.BlockSpec((BM, D_OUT), lambda i: (i, _z0(i))),
        out_shape=jax.ShapeDtypeStruct((N_NODES, D_OUT), jnp.float32),
    )(s, x, wcat, bias2d)


def kernel(x, edge_index_r0, edge_index_r1, edge_index_r2, edge_index_r3,
           w, coeff_mat, W_self, bias):
    x = x.astype(jnp.float32)
    xa = jnp.concatenate(
        [x, jnp.ones((N_NODES, 1), jnp.float32),
         jnp.zeros((N_NODES, DW - D_IN - 1), jnp.float32)], axis=1)

    srcs, dsts = [], []
    for e in (edge_index_r0, edge_index_r1, edge_index_r2, edge_index_r3):
        src = e[0].astype(jnp.int32)
        dst = e[1].astype(jnp.int32)
        srcs.append(jnp.concatenate(
            [src, jnp.zeros((E_PAD - E,), jnp.int32)]))
        dsts.append(jnp.concatenate(
            [dst, jnp.full((E_PAD - E,), N_NODES, jnp.int32)]))
    src = jnp.stack(srcs)
    dst = jnp.stack(dsts)

    rsrc, rldst, rcnt = _route(src, dst)
    s = _sc_aggregate(xa, rsrc, rldst, rcnt)

    w2d = w.astype(jnp.float32).reshape(w.shape[0], D_IN * D_OUT)
    wcat = _combine_wcat(coeff_mat.astype(jnp.float32), w2d,
                         W_self.astype(jnp.float32))
    bias2d = bias.astype(jnp.float32).reshape(1, D_OUT)
    return _tc_combine(s, x, wcat, bias2d)


# speedup vs baseline: 1.5349x; 1.5349x over previous
import functools

import jax
import jax.numpy as jnp
from jax import lax
from jax.experimental import pallas as pl
from jax.experimental.pallas import tpu as pltpu
from jax.experimental.pallas import tpu_sc as plsc

N_NODES = 10000
N_REL = 4
D_IN = 256
D_OUT = 256
DW = 272
E = 40000
LANES = 16
NC = 2
NS = 16
NW = NC * NS
E_WORK = 1280
E_PAD = NW * E_WORK
LCAP = E_WORK + 64
SEG = E_WORK
CHUNK = 64
KMAX = SEG // CHUNK
HALF = N_NODES // NC
ACC_ROWS = 5024
OUT_UNIT = 40
N_UNITS = HALF // OUT_UNIT


def _scalar(v16):
    return lax.reduce_max(v16, axes=(0,))


def _router_body(src_hbm, dst_hbm, rsrc_hbm, rldst_hbm, rcnt_hbm,
                 sv_all, dv_all, l0s, l0d, l1s, l1d, cbuf):
    i32 = jnp.int32
    cid = lax.axis_index("c").astype(i32)
    sid = lax.axis_index("s").astype(i32)
    w = cid * i32(NS) + sid

    for r in range(N_REL):
        r = i32(r)
        ebase = w * i32(E_WORK)
        pltpu.sync_copy(src_hbm.at[r, pl.ds(ebase, E_WORK)], sv_all)
        pltpu.sync_copy(dst_hbm.at[r, pl.ds(ebase, E_WORK)], dv_all)

        def step(i, carry):
            c0, c1 = carry
            i = i.astype(jnp.int32)
            sv = sv_all[pl.ds(i * LANES, LANES)]
            dv = dv_all[pl.ds(i * LANES, LANES)]
            m0 = dv < i32(HALF)
            m1 = (dv >= i32(HALF)) & (dv < i32(N_NODES))
            inc0 = plsc.cumsum(jnp.where(m0, i32(1), i32(0)))
            inc1 = plsc.cumsum(jnp.where(m1, i32(1), i32(0)))
            pos0 = jnp.maximum(c0 + inc0 - i32(1), i32(0))
            pos1 = jnp.maximum(c1 + inc1 - i32(1), i32(0))
            plsc.store_scatter(l0s, [pos0], sv, mask=m0)
            plsc.store_scatter(l0d, [pos0], dv, mask=m0)
            plsc.store_scatter(l1s, [pos1], sv, mask=m1)
            plsc.store_scatter(l1d, [pos1], dv - i32(HALF), mask=m1)
            n0 = _scalar(inc0)
            n1 = _scalar(inc1)
            return c0 + n0, c1 + n1

        c0, c1 = pl.loop(i32(0), i32(E_WORK // LANES),
                         init_carry=(i32(0), i32(0)))(step)

        trash = i32(HALF) + (w % i32(NS))
        for t in range(4):
            off = i32(t * LANES)
            zs = jnp.zeros((LANES,), jnp.int32)
            ts = jnp.full((LANES,), 0, jnp.int32) + trash
            l0s[pl.ds(c0 + off, LANES)] = zs
            l0d[pl.ds(c0 + off, LANES)] = ts
            l1s[pl.ds(c1 + off, LANES)] = zs
            l1d[pl.ds(c1 + off, LANES)] = ts
        c0p = ((c0 + i32(63)) // i32(64)) * i32(64)
        c1p = ((c1 + i32(63)) // i32(64)) * i32(64)

        @pl.loop(i32(0), i32(KMAX))
        def _f(k):
            k = k.astype(jnp.int32)

            @pl.when(k * i32(CHUNK) < c0p)
            def _():
                pltpu.sync_copy(l0s.at[pl.ds(k * CHUNK, CHUNK)],
                                rsrc_hbm.at[r, i32(0), w, k])
                pltpu.sync_copy(l0d.at[pl.ds(k * CHUNK, CHUNK)],
                                rldst_hbm.at[r, i32(0), w, k])

            @pl.when(k * i32(CHUNK) < c1p)
            def _():
                pltpu.sync_copy(l1s.at[pl.ds(k * CHUNK, CHUNK)],
                                rsrc_hbm.at[r, i32(1), w, k])
                pltpu.sync_copy(l1d.at[pl.ds(k * CHUNK, CHUNK)],
                                rldst_hbm.at[r, i32(1), w, k])

        cbuf[pl.ds(i32(0), LANES)] = jnp.zeros((LANES,), jnp.int32) + c0p
        pltpu.sync_copy(cbuf, rcnt_hbm.at[r, i32(0), w])
        cbuf[pl.ds(i32(0), LANES)] = jnp.zeros((LANES,), jnp.int32) + c1p
        pltpu.sync_copy(cbuf, rcnt_hbm.at[r, i32(1), w])


@jax.jit
def _route(src, dst):
    mesh = plsc.VectorSubcoreMesh(core_axis_name="c", subcore_axis_name="s")
    f = pl.kernel(
        _router_body,
        out_type=(
            jax.ShapeDtypeStruct((N_REL, NC, NW, KMAX, CHUNK), jnp.int32),
            jax.ShapeDtypeStruct((N_REL, NC, NW, KMAX, CHUNK), jnp.int32),
            jax.ShapeDtypeStruct((N_REL, NC, NW, LANES), jnp.int32),
        ),
        mesh=mesh,
        scratch_types=[
            pltpu.VMEM((E_WORK,), jnp.int32),
            pltpu.VMEM((E_WORK,), jnp.int32),
            pltpu.VMEM((LCAP,), jnp.int32),
            pltpu.VMEM((LCAP,), jnp.int32),
            pltpu.VMEM((LCAP,), jnp.int32),
            pltpu.VMEM((LCAP,), jnp.int32),
            pltpu.VMEM((LANES,), jnp.int32),
        ],
        compiler_params=pltpu.CompilerParams(use_tc_tiling_on_sc=False,
                                             needs_layout_passes=False),
    )
    return f(src, dst)


def _agg_body(xa_hbm, rsrc_hbm, rldst_hbm, rcnt_hbm, out_hbm,
              acc_sh, sidx, lidx, rows2, zrow_v, cbuf, sems):
    i32 = jnp.int32
    cid = lax.axis_index("c").astype(i32)
    sid = lax.axis_index("s").astype(i32)
    base_dst = cid * i32(HALF)

    @pl.loop(i32(0), i32(DW // LANES))
    def _zb(j):
        j = j.astype(jnp.int32)
        for i in range(8):
            zrow_v[i, pl.ds(j * LANES, LANES)] = jnp.zeros((LANES,), jnp.float32)

    for r in range(N_REL):
        r = i32(r)

        @pl.loop(i32(0), i32((ACC_ROWS // 8 + NS - 1) // NS))
        def _z(k):
            zu = sid + k.astype(jnp.int32) * i32(NS)

            @pl.when(zu < i32(ACC_ROWS // 8))
            def _():
                pltpu.sync_copy(zrow_v, acc_sh.at[pl.ds(zu * 8, 8)])

        plsc.subcore_barrier()

        for seg in range(2):
            w = sid + i32(seg * NS)
            pltpu.sync_copy(rcnt_hbm.at[r, cid, w], cbuf)
            n = _scalar(cbuf[pl.ds(i32(0), LANES)])
            nch = n // i32(CHUNK)
            pltpu.sync_copy(rsrc_hbm.at[r, cid, w], sidx)
            pltpu.sync_copy(rldst_hbm.at[r, cid, w], lidx)

            def fetch(k, b):
                pltpu.async_copy(xa_hbm.at[sidx.at[k]],
                                 rows2.at[i32(b)], sems.at[i32(b)])

            def wait_gather(b):
                pltpu.make_async_copy(
                    xa_hbm.at[pl.ds(i32(0), CHUNK)],
                    rows2.at[i32(b)], sems.at[i32(b)]).wait()

            @pl.when(i32(0) < nch)
            def _():
                fetch(i32(0), 0)

            @pl.when(i32(1) < nch)
            def _():
                fetch(i32(1), 1)

            @pl.loop(i32(0), i32(KMAX), step=i32(2))
            def _e(j):
                j = j.astype(jnp.int32)
                for b in range(2):
                    jj = j + i32(b)

                    @pl.when(jj < nch)
                    def _():
                        wait_gather(b)
                        pltpu.sync_copy(rows2.at[i32(b)],
                                        acc_sh.at[lidx.at[jj]], add=True)

                        @pl.when(jj + i32(2) < nch)
                        def _():
                            fetch(jj + i32(2), b)

        plsc.subcore_barrier()

        @pl.loop(i32(0), i32((N_UNITS + NS - 1) // NS))
        def _w(k):
            u = sid + k.astype(jnp.int32) * i32(NS)

            @pl.when(u < i32(N_UNITS))
            def _():
                pltpu.sync_copy(
                    acc_sh.at[pl.ds(u * OUT_UNIT, OUT_UNIT)],
                    out_hbm.at[r, pl.ds(base_dst + u * OUT_UNIT, OUT_UNIT)])

        plsc.subcore_barrier()


@jax.jit
def _sc_aggregate(xa, rsrc, rldst, rcnt):
    mesh = plsc.VectorSubcoreMesh(core_axis_name="c", subcore_axis_name="s")
    f = pl.kernel(
        _agg_body,
        out_type=jax.ShapeDtypeStruct((N_REL, N_NODES, DW), jnp.float32),
        mesh=mesh,
        scratch_types=[
            pltpu.VMEM_SHARED((ACC_ROWS, DW), jnp.float32),
            pltpu.VMEM((KMAX, CHUNK), jnp.int32),
            pltpu.VMEM((KMAX, CHUNK), jnp.int32),
            pltpu.VMEM((2, CHUNK, DW), jnp.float32),
            pltpu.VMEM((8, DW), jnp.float32),
            pltpu.VMEM((LANES,), jnp.int32),
            pltpu.SemaphoreType.DMA((2,)),
        ],
        compiler_params=pltpu.CompilerParams(use_tc_tiling_on_sc=False,
                                             needs_layout_passes=False),
    )
    return f(xa, rsrc, rldst, rcnt)


def _wcat_body(coeff_ref, w2d_ref, wself_ref, o_ref):
    wall = jnp.dot(coeff_ref[...], w2d_ref[...],
                   preferred_element_type=jnp.float32,
                   precision=jax.lax.Precision.HIGHEST)
    o_ref[: N_REL * D_IN, :] = wall.reshape(N_REL * D_IN, D_OUT)
    o_ref[N_REL * D_IN:, :] = wself_ref[...]


def _combine_wcat(coeff_mat, w2d, W_self):
    return pl.pallas_call(
        _wcat_body,
        out_shape=jax.ShapeDtypeStruct(((N_REL + 1) * D_IN, D_OUT), jnp.float32),
    )(coeff_mat, w2d, W_self)


def _z0(i):
    return i * 0


BM = 400


def _tc_body(s_ref, x_ref, wcat_ref, bias_ref, o_ref):
    parts = []
    for r in range(N_REL):
        cnt = s_ref[r, :, D_IN:D_IN + 1]
        sm = s_ref[r, :, :D_IN]
        parts.append(jnp.where(cnt > 0, sm / jnp.maximum(cnt, 1.0), 0.0))
    parts.append(x_ref[...])
    xin = jnp.concatenate(parts, axis=1)
    acc = jnp.dot(xin, wcat_ref[...], preferred_element_type=jnp.float32)
    o_ref[...] = jnp.maximum(acc + bias_ref[...], 0.0)


def _tc_combine(s, x, wcat, bias2d):
    grid = (N_NODES // BM,)
    return pl.pallas_call(
        _tc_body,
        grid=grid,
        in_specs=[
            pl.BlockSpec((N_REL, BM, DW), lambda i: (_z0(i), i, _z0(i))),
            pl.BlockSpec((BM, D_IN), lambda i: (i, _z0(i))),
            pl.BlockSpec(((N_REL + 1) * D_IN, D_OUT),
                         lambda i: (_z0(i), _z0(i))),
            pl.BlockSpec((1, D_OUT), lambda i: (_z0(i), _z0(i))),
        ],
        out_specs=pl.BlockSpec((BM, D_OUT), lambda i: (i, _z0(i))),
        out_shape=jax.ShapeDtypeStruct((N_NODES, D_OUT), jnp.float32),
    )(s, x, wcat, bias2d)


def kernel(x, edge_index_r0, edge_index_r1, edge_index_r2, edge_index_r3,
           w, coeff_mat, W_self, bias):
    x = x.astype(jnp.float32)
    xa = jnp.concatenate(
        [x, jnp.ones((N_NODES, 1), jnp.float32),
         jnp.zeros((N_NODES, DW - D_IN - 1), jnp.float32)], axis=1)

    srcs, dsts = [], []
    for e in (edge_index_r0, edge_index_r1, edge_index_r2, edge_index_r3):
        src = e[0].astype(jnp.int32)
        dst = e[1].astype(jnp.int32)
        srcs.append(jnp.concatenate(
            [src, jnp.zeros((E_PAD - E,), jnp.int32)]))
        dsts.append(jnp.concatenate(
            [dst, jnp.full((E_PAD - E,), N_NODES, jnp.int32)]))
    src = jnp.stack(srcs)
    dst = jnp.stack(dsts)

    rsrc, rldst, rcnt = _route(src, dst)
    s = _sc_aggregate(xa, rsrc, rldst, rcnt)

    w2d = w.astype(jnp.float32).reshape(w.shape[0], D_IN * D_OUT)
    wcat = _combine_wcat(coeff_mat.astype(jnp.float32), w2d,
                         W_self.astype(jnp.float32))
    bias2d = bias.astype(jnp.float32).reshape(1, D_OUT)
    return _tc_combine(s, x, wcat, bias2d)
